# baseline (speedup 1.0000x reference)
import functools

import jax
import jax.numpy as jnp
from jax import lax
from jax.experimental import pallas as pl
from jax.experimental.pallas import tpu as pltpu
from jax.experimental.pallas import tpu_sc as plsc

B = 4096
L = 200
H = 64
OUT = 16
V = 1000000

NC = 2
NS = 16
NW = NC * NS
BPW = B // NW
NSPLIT = 2
LH = L // NSPLIT
UNROLL = 4
NBUF = 4

CB = 16384


def _mm_body(e_ref, w_ref, o_ref):
    d = lax.dot_general(
        e_ref[...], w_ref[...],
        dimension_numbers=(((0,), (1,)), ((), ())),
        preferred_element_type=jnp.float32,
    )
    o_ref[...] = d.astype(jnp.bfloat16)


def _tc_fold(et, W):
    wide = pl.pallas_call(
        _mm_body,
        grid=(pl.cdiv(V, CB),),
        in_specs=[
            pl.BlockSpec((H, CB), lambda i: (0, i)),
            pl.BlockSpec((128, H), lambda i: (0, 0)),
        ],
        out_specs=pl.BlockSpec((CB, 128), lambda i: (i, 0)),
        out_shape=jax.ShapeDtypeStruct((V, 128), jnp.bfloat16),
    )(et, W)
    return lax.bitcast_convert_type(
        wide.reshape(V * 4, OUT, 2), jnp.int32)


def _issue_gathers(tab_hbm, idx_all, rows_v, sem, row, buf):
    for h in range(NSPLIT):
        pltpu.make_async_copy(
            tab_hbm.at[idx_all.at[row, h]], rows_v.at[buf, h], sem
        ).start()


def _wait_gathers(tab_hbm, idx_all, rows_v, sem, row, buf):
    for h in range(NSPLIT):
        pltpu.make_async_copy(
            tab_hbm.at[idx_all.at[row, h]], rows_v.at[buf, h], sem
        ).wait()


def _reduce_row(rows_v, acc_v, tmp_v, bias, i, buf):
    def body(j, accs):
        ae, ao = accs
        for u in range(UNROLL):
            for h in range(NSPLIT):
                bits = rows_v[buf, h, UNROLL * j + u, pl.ds(0, 16)]
                ae = ae + lax.bitcast_convert_type(
                    lax.shift_left(bits, jnp.int32(16)), jnp.float32)
                ao = ao + lax.bitcast_convert_type(
                    lax.bitwise_and(bits, jnp.int32(-65536)), jnp.float32)
        return ae, ao

    zero = jnp.zeros((16,), jnp.float32)
    ae, ao = lax.fori_loop(0, LH // UNROLL, body, (zero, zero))
    tmp_v[pl.ds(0, 16)] = ae
    tmp_v[pl.ds(8, 16)] = ao
    acc_v[i, pl.ds(0, 16)] = tmp_v[pl.ds(0, 16)] + bias


def _sc_pool_body(x_hbm, tab_hbm, b_hbm, out_hbm,
                  idx_all, rows_v, acc_v, tmp_v, b_v, sem0, sem1, sem2, sem3):
    wid = lax.axis_index("s") * NC + lax.axis_index("c")
    base = wid * BPW

    pltpu.sync_copy(x_hbm.at[pl.ds(base, BPW)], idx_all)
    pltpu.sync_copy(b_hbm, b_v)
    bias = b_v[pl.ds(0, 16)]

    sems = (sem0, sem1, sem2, sem3)
    for bufi in range(NBUF):
        _issue_gathers(tab_hbm, idx_all, rows_v, sems[bufi], bufi, bufi)

    def ring_body(g, _):
        row = NBUF * g
        for bufi in range(NBUF):
            r = row + bufi
            _wait_gathers(tab_hbm, idx_all, rows_v, sems[bufi], r, bufi)

            @pl.when(r + NBUF < BPW)
            def _():
                _issue_gathers(tab_hbm, idx_all, rows_v, sems[bufi],
                               r + NBUF, bufi)

            _reduce_row(rows_v, acc_v, tmp_v, bias, r, bufi)
        return 0

    lax.fori_loop(0, BPW // NBUF, ring_body, 0)

    pltpu.sync_copy(acc_v, out_hbm.at[pl.ds(base, BPW)])


@functools.partial(
    pl.kernel,
    mesh=plsc.VectorSubcoreMesh(core_axis_name="c", subcore_axis_name="s"),
    compiler_params=pltpu.CompilerParams(use_tc_tiling_on_sc=False),
    out_type=jax.ShapeDtypeStruct((B, OUT), jnp.float32),
    scratch_types=[
        pltpu.VMEM((BPW, NSPLIT, LH), jnp.int32),
        pltpu.VMEM((NBUF, NSPLIT, LH, OUT), jnp.int32),
        pltpu.VMEM((BPW, OUT), jnp.float32),
        pltpu.VMEM((32,), jnp.float32),
        pltpu.VMEM((16,), jnp.float32),
        pltpu.SemaphoreType.DMA,
        pltpu.SemaphoreType.DMA,
        pltpu.SemaphoreType.DMA,
        pltpu.SemaphoreType.DMA,
    ],
)
def _sc_pool(x_hbm, tab_hbm, b_hbm, out_hbm,
             idx_all, rows_v, acc_v, tmp_v, b_v, sem0, sem1, sem2, sem3):
    _sc_pool_body(x_hbm, tab_hbm, b_hbm, out_hbm,
                  idx_all, rows_v, acc_v, tmp_v, b_v, sem0, sem1, sem2, sem3)


def kernel(x, emb, W, b):
    x32 = (x.astype(jnp.int32) * 4).reshape(B, NSPLIT, LH)
    inter = jnp.stack(
        [jnp.arange(8, dtype=jnp.int32),
         jnp.arange(8, 16, dtype=jnp.int32)], axis=1).reshape(16)
    wp = jnp.zeros((128, H), jnp.float32).at[:OUT].set(
        (W * (1.0 / L))[inter])
    tab = _tc_fold(emb.T, wp)
    return _sc_pool(x32, tab, b)


# speedup vs baseline: 313.5138x; 313.5138x over previous
import functools

import jax
import jax.numpy as jnp
from jax import lax
from jax.experimental import pallas as pl
from jax.experimental.pallas import tpu as pltpu
from jax.experimental.pallas import tpu_sc as plsc

B = 4096
L = 200
H = 64
OUT = 16
V = 1000000

NC = 2
NS = 16
NW = NC * NS
BPW = B // NW
NSPLIT = 2
LH = L // NSPLIT
UNROLL = 4
NBUF = 4

CB = 16384


def _mm_body(e_ref, w_ref, o_ref):
    wz = jnp.concatenate(
        [w_ref[...] * (1.0 / L), jnp.zeros((128 - OUT, H), jnp.float32)],
        axis=0,
    )
    o_ref[...] = lax.dot_general(
        e_ref[...], wz,
        dimension_numbers=(((0,), (1,)), ((), ())),
        preferred_element_type=jnp.float32,
    )


def _tc_fold(et, W):
    wide = pl.pallas_call(
        _mm_body,
        grid=(pl.cdiv(V, CB),),
        in_specs=[
            pl.BlockSpec((H, CB), lambda i: (0, i)),
            pl.BlockSpec((OUT, H), lambda i: (0, 0)),
        ],
        out_specs=pl.BlockSpec((CB, 128), lambda i: (i, 0)),
        out_shape=jax.ShapeDtypeStruct((V, 128), jnp.float32),
    )(et, W)
    return wide.reshape(V * 8, OUT)


def _issue_gathers(tab_hbm, idx_all, rows_v, sem, row, buf):
    for h in range(NSPLIT):
        pltpu.make_async_copy(
            tab_hbm.at[idx_all.at[row, h]], rows_v.at[buf, h], sem
        ).start()


def _wait_gathers(tab_hbm, idx_all, rows_v, sem, row, buf):
    for h in range(NSPLIT):
        pltpu.make_async_copy(
            tab_hbm.at[idx_all.at[row, h]], rows_v.at[buf, h], sem
        ).wait()


def _reduce_row(rows_v, acc_v, bias, i, buf):
    def body(j, a):
        for u in range(UNROLL):
            for h in range(NSPLIT):
                a = a + rows_v[buf, h, UNROLL * j + u, pl.ds(0, 16)]
        return a

    acc = lax.fori_loop(0, LH // UNROLL, body, bias)
    acc_v[i, pl.ds(0, 16)] = acc


def _sc_pool_body(x_hbm, tab_hbm, b_hbm, out_hbm,
                  idx_all, rows_v, acc_v, b_v, sem0, sem1, sem2, sem3):
    wid = lax.axis_index("s") * NC + lax.axis_index("c")
    base = wid * BPW

    pltpu.sync_copy(x_hbm.at[pl.ds(base, BPW)], idx_all)
    pltpu.sync_copy(b_hbm, b_v)
    bias = b_v[pl.ds(0, 16)]

    sems = (sem0, sem1, sem2, sem3)
    for bufi in range(NBUF):
        _issue_gathers(tab_hbm, idx_all, rows_v, sems[bufi], bufi, bufi)

    def ring_body(g, _):
        row = NBUF * g
        for bufi in range(NBUF):
            r = row + bufi
            _wait_gathers(tab_hbm, idx_all, rows_v, sems[bufi], r, bufi)

            @pl.when(r + NBUF < BPW)
            def _():
                _issue_gathers(tab_hbm, idx_all, rows_v, sems[bufi],
                               r + NBUF, bufi)

            _reduce_row(rows_v, acc_v, bias, r, bufi)
        return 0

    lax.fori_loop(0, BPW // NBUF, ring_body, 0)

    pltpu.sync_copy(acc_v, out_hbm.at[pl.ds(base, BPW)])


@functools.partial(
    pl.kernel,
    mesh=plsc.VectorSubcoreMesh(core_axis_name="c", subcore_axis_name="s"),
    compiler_params=pltpu.CompilerParams(use_tc_tiling_on_sc=False),
    out_type=jax.ShapeDtypeStruct((B, OUT), jnp.float32),
    scratch_types=[
        pltpu.VMEM((BPW, NSPLIT, LH), jnp.int32),
        pltpu.VMEM((NBUF, NSPLIT, LH, OUT), jnp.float32),
        pltpu.VMEM((BPW, OUT), jnp.float32),
        pltpu.VMEM((16,), jnp.float32),
        pltpu.SemaphoreType.DMA,
        pltpu.SemaphoreType.DMA,
        pltpu.SemaphoreType.DMA,
        pltpu.SemaphoreType.DMA,
    ],
)
def _sc_pool(x_hbm, tab_hbm, b_hbm, out_hbm,
             idx_all, rows_v, acc_v, b_v, sem0, sem1, sem2, sem3):
    _sc_pool_body(x_hbm, tab_hbm, b_hbm, out_hbm,
                  idx_all, rows_v, acc_v, b_v, sem0, sem1, sem2, sem3)


def kernel(x, emb, W, b):
    x32 = (x.astype(jnp.int32) * 8).reshape(B, NSPLIT, LH)
    tab = _tc_fold(emb.T, W)
    return _sc_pool(x32, tab, b)
